# baseline (speedup 1.0000x reference)
import jax
import jax.numpy as jnp
from jax import lax
from jax.experimental import pallas as pl
from jax.experimental.pallas import tpu as pltpu
from jax.experimental.pallas import tpu_sc as plsc

NC = 2
NS = 16
L = 16
NW = NC * NS


def _mesh():
    return plsc.VectorSubcoreMesh(core_axis_name="c", subcore_axis_name="s")


def _make_k1(T, N, RP, NT_PAD, CH, WIN):
    NRP = RP * N
    ZSL = NRP // NS

    def body(rows_h, cols_h, fr_h, to_h, rdiag_h,
             subj_h, obj_h, degkey_h, w1key_h, deg_h,
             rows_v, cols_v, frw_v, tow_v, sv_v, ov_v, dk_v, wk_v, val_v,
             rdiag_v, zero_v, deg_s):
        cid = lax.axis_index("c")
        sid = lax.axis_index("s")
        wid = sid * NC + cid
        t0 = wid * CH
        pltpu.sync_copy(rows_h.at[pl.ds(t0, CH)], rows_v)
        pltpu.sync_copy(cols_h.at[pl.ds(t0, CH)], cols_v)
        pltpu.sync_copy(rdiag_h, rdiag_v)

        def _z(i, c):
            zero_v[pl.ds(i * L, L)] = jnp.zeros((L,), jnp.float32)
            return c
        lax.fori_loop(0, ZSL // L, _z, 0)
        pltpu.sync_copy(zero_v, deg_s.at[pl.ds(sid * ZSL, ZSL)])

        base = rows_v[pl.ds(0, L)][0]
        base_al = (base // 8) * 8
        pltpu.sync_copy(fr_h.at[pl.ds(base_al, WIN)], frw_v)
        pltpu.sync_copy(to_h.at[pl.ds(base_al, WIN)], tow_v)

        def _g(g, c):
            o = g * L
            idx = rows_v[pl.ds(o, L)] - base_al
            sv = plsc.load_gather(frw_v, [idx])
            ov = plsc.load_gather(tow_v, [idx])
            cv = cols_v[pl.ds(o, L)]
            vv = plsc.load_gather(rdiag_v, [cv])
            sv_v[0, pl.ds(o, L)] = sv
            ov_v[0, pl.ds(o, L)] = ov
            dk_v[0, pl.ds(o, L)] = cv * N + sv
            wk_v[0, pl.ds(o, L)] = cv * N + ov
            val_v[0, pl.ds(o, L)] = vv
            return c
        lax.fori_loop(0, CH // L, _g, 0)

        pltpu.sync_copy(sv_v.at[0], subj_h.at[pl.ds(t0, CH)])
        pltpu.sync_copy(ov_v.at[0], obj_h.at[pl.ds(t0, CH)])
        pltpu.sync_copy(dk_v.at[0], degkey_h.at[pl.ds(t0, CH)])
        pltpu.sync_copy(wk_v.at[0], w1key_h.at[pl.ds(t0, CH)])

        plsc.subcore_barrier()
        pltpu.sync_copy(val_v.at[0], deg_s.at[dk_v.at[0]], add=True)
        plsc.subcore_barrier()

        @pl.when(sid == 0)
        def _():
            pltpu.sync_copy(deg_s, deg_h.at[cid])

    i32, f32 = jnp.int32, jnp.float32
    return pl.kernel(
        body,
        out_type=(
            jax.ShapeDtypeStruct((T,), i32),
            jax.ShapeDtypeStruct((T,), i32),
            jax.ShapeDtypeStruct((T,), i32),
            jax.ShapeDtypeStruct((T,), i32),
            jax.ShapeDtypeStruct((NC, NRP), f32),
        ),
        mesh=_mesh(),
        compiler_params=pltpu.CompilerParams(needs_layout_passes=False, use_tc_tiling_on_sc=False),
        scratch_types=[
            pltpu.VMEM((CH,), i32),
            pltpu.VMEM((CH,), i32),
            pltpu.VMEM((WIN,), i32),
            pltpu.VMEM((WIN,), i32),
            pltpu.VMEM((1, CH), i32),
            pltpu.VMEM((1, CH), i32),
            pltpu.VMEM((1, CH), i32),
            pltpu.VMEM((1, CH), i32),
            pltpu.VMEM((1, CH), f32),
            pltpu.VMEM((L,), f32),
            pltpu.VMEM((ZSL,), f32),
            pltpu.VMEM_SHARED((RP * N,), f32),
        ],
    )


def _make_k1c(N, RP):
    def body(deg_ref, rd_ref, o_ref):
        d = deg_ref[0] + deg_ref[1]
        o_ref[...] = jnp.where(d > 0.0, rd_ref[...] / d, 0.0)

    return pl.pallas_call(
        body,
        in_specs=[
            pl.BlockSpec((NC, RP, N), lambda: (0, 0, 0)),
            pl.BlockSpec((RP, 1), lambda: (0, 0)),
        ],
        out_specs=pl.BlockSpec((RP, N), lambda: (0, 0)),
        out_shape=jax.ShapeDtypeStruct((RP, N), jnp.float32),
    )


def _make_k2(T, N, RP, E, CH, SB):
    NSUB = CH // SB
    RPS = N // NS

    def body(subj_h, w1key_h, degkey_h, scale_h, w1_h,
             hpart_h,
             sv0, sv1, wk0, wk1, dk0, dk1, sc0, sc1, rw0, rw1, zero_v, h_s,
             semr0, semr1, sems0, sems1):
        cid = lax.axis_index("c")
        sid = lax.axis_index("s")
        wid = sid * NC + cid
        t0 = wid * CH
        iota = lax.iota(jnp.int32, L)
        sv = [sv0, sv1]
        wk = [wk0, wk1]
        dk = [dk0, dk1]
        scb = [sc0, sc1]
        rw = [rw0, rw1]
        semr = [semr0, semr1]
        sems = [sems0, sems1]

        def _z(i, c):
            zero_v[i, :] = jnp.zeros((L,), jnp.float32)
            return c
        lax.fori_loop(0, RPS, _z, 0)
        pltpu.sync_copy(zero_v, h_s.at[pl.ds(sid * RPS, RPS)])
        plsc.subcore_barrier()

        def _issue(i, b):
            off = t0 + i * SB
            pltpu.sync_copy(w1key_h.at[pl.ds(off, SB)], wk[b].at[0])
            pltpu.sync_copy(degkey_h.at[pl.ds(off, SB)], dk[b].at[0])
            pltpu.sync_copy(subj_h.at[pl.ds(off, SB)], sv[b].at[0])
            return (pltpu.async_copy(w1_h.at[wk[b].at[0]], rw[b], semr[b]),
                    pltpu.async_copy(scale_h.at[dk[b].at[0]], scb[b], sems[b]))

        pend = {0: _issue(0, 0)}
        for sc in range(NSUB):
            b = sc % 2
            for cp in pend.pop(sc):
                cp.wait()
            if sc + 1 < NSUB:
                pend[sc + 1] = _issue(sc + 1, (sc + 1) % 2)

            def _m(g, c):
                o = g * L
                s = scb[b][pl.ds(o, L)]
                ridx = o + iota
                for j in range(E):
                    jv = jnp.full((L,), j, dtype=jnp.int32)
                    col = plsc.load_gather(rw[b], [ridx, jv])
                    plsc.store_scatter(rw[b], [ridx, jv], col * s)
                return c
            lax.fori_loop(0, SB // L, _m, 0)
            pltpu.sync_copy(rw[b], h_s.at[sv[b].at[0]], add=True)

        plsc.subcore_barrier()

        @pl.when(sid == 0)
        def _():
            pltpu.sync_copy(h_s, hpart_h.at[cid])

    i32, f32 = jnp.int32, jnp.float32
    return pl.kernel(
        body,
        out_type=jax.ShapeDtypeStruct((NC, N, E), f32),
        mesh=_mesh(),
        compiler_params=pltpu.CompilerParams(needs_layout_passes=False, use_tc_tiling_on_sc=False),
        scratch_types=[
            pltpu.VMEM((1, SB), i32),
            pltpu.VMEM((1, SB), i32),
            pltpu.VMEM((1, SB), i32),
            pltpu.VMEM((1, SB), i32),
            pltpu.VMEM((1, SB), i32),
            pltpu.VMEM((1, SB), i32),
            pltpu.VMEM((SB,), f32),
            pltpu.VMEM((SB,), f32),
            pltpu.VMEM((SB, E), f32),
            pltpu.VMEM((SB, E), f32),
            pltpu.VMEM((RPS, E), f32),
            pltpu.VMEM_SHARED((N, E), f32),
            pltpu.SemaphoreType.DMA,
            pltpu.SemaphoreType.DMA,
            pltpu.SemaphoreType.DMA,
            pltpu.SemaphoreType.DMA,
        ],
    )


def _make_k2b(N, E, NCOLS):
    NR = N * E // NCOLS

    def body(p_ref, b_ref, o_ref):
        o_ref[...] = jnp.maximum(p_ref[0] + p_ref[1] + b_ref[...], 0.0)

    return pl.pallas_call(
        body,
        out_shape=jax.ShapeDtypeStruct((NR, NCOLS), jnp.float32),
    )


def _make_k3(T, N, RP, E, SB, TR, BLK):
    NQ = 4
    QN = (RP // NQ) * N
    CH3 = T // NS
    NSUB = CH3 // SB
    ZR = (QN + TR) // NS
    ZB = ZR // 4
    CSB = SB + BLK + 2 * L

    def body(obj_h, degkey_h, hrelu_h,
             s2_h,
             ob_v, dk_v, cob_v, clk_v, blk_v, hrows_v, zero_v, s2_s, sem):
        cid = lax.axis_index("c")
        sid = lax.axis_index("s")
        iota = lax.iota(jnp.int32, L)

        for p in range(NQ // NC):
            q = NC * p + cid
            rbase = q * QN

            def _z(i, c):
                zero_v[i, :] = jnp.zeros((L,), jnp.float32)
                return c
            lax.fori_loop(0, ZB, _z, 0)
            for z in range(4):
                pltpu.sync_copy(zero_v, s2_s.at[pl.ds(sid * ZR + z * ZB, ZB)])
            plsc.subcore_barrier()

            for sc in range(NSUB):
                off = sid * CH3 + sc * SB
                pltpu.sync_copy(obj_h.at[pl.ds(off, SB)], ob_v.at[0])
                pltpu.sync_copy(degkey_h.at[pl.ds(off, SB)], dk_v.at[0])

                def _cg(g, cnt):
                    o = g * L
                    kk = dk_v[0, pl.ds(o, L)] - rbase
                    m = (kk >= 0) & (kk < QN)
                    ov = ob_v[0, pl.ds(o, L)]
                    plsc.store_compressed(cob_v.at[pl.ds(cnt, L)], ov, mask=m)
                    plsc.store_compressed(clk_v.at[pl.ds(cnt, L)], kk, mask=m)
                    return cnt + plsc.all_reduce_population_count(m)[0]
                cnt = lax.fori_loop(0, SB // L, _cg, 0)

                def _pad(i, c):
                    o2 = cnt + i * L
                    cob_v[pl.ds(o2, L)] = (o2 + iota) & (L - 1)
                    clk_v[pl.ds(o2, L)] = QN + ((o2 + iota) & (TR - 1))
                    return c
                lax.fori_loop(0, BLK // L + 1, _pad, 0)

                nblk = (cnt + BLK - 1) // BLK

                def _bl(b2, c):
                    o2 = b2 * BLK

                    def _cp(i, c2):
                        blk_v[0, pl.ds(i * L, L)] = clk_v[pl.ds(o2 + i * L, L)]
                        return c2
                    lax.fori_loop(0, BLK // L, _cp, 0)
                    pltpu.async_copy(hrelu_h.at[cob_v.at[pl.ds(o2, BLK)]],
                                     hrows_v, sem).wait()
                    pltpu.sync_copy(hrows_v, s2_s.at[blk_v.at[0]], add=True)
                    return c
                lax.fori_loop(0, nblk, _bl, 0)

            plsc.subcore_barrier()

            @pl.when(sid == 0)
            def _():
                pltpu.sync_copy(s2_s.at[pl.ds(0, QN)], s2_h.at[q])
            plsc.subcore_barrier()

    i32, f32 = jnp.int32, jnp.float32
    return pl.kernel(
        body,
        out_type=jax.ShapeDtypeStruct((NQ, QN, E), f32),
        mesh=_mesh(),
        compiler_params=pltpu.CompilerParams(needs_layout_passes=False, use_tc_tiling_on_sc=False),
        scratch_types=[
            pltpu.VMEM((1, SB), i32),
            pltpu.VMEM((1, SB), i32),
            pltpu.VMEM((CSB,), i32),
            pltpu.VMEM((CSB,), i32),
            pltpu.VMEM((1, BLK), i32),
            pltpu.VMEM((BLK, E), f32),
            pltpu.VMEM((ZB, E), f32),
            pltpu.VMEM_SHARED((QN + TR, E), f32),
            pltpu.SemaphoreType.DMA,
        ],
    )


def _make_k4(N, RP, E, C, NB):
    NQ = 4
    RQ = RP // NQ
    GRID = N // NB

    def body(s2_ref, deg_ref, rd_ref, w2_ref, b2_ref, o_ref):
        d = deg_ref[:, :RP] + deg_ref[:, RP:]
        scale = jnp.where(d > 0.0, rd_ref[...] / d, 0.0)
        acc = jnp.zeros((NB, C), jnp.float32)
        for r in range(RP):
            h2r = s2_ref[r // RQ, r % RQ] * scale[:, r][:, None]
            acc += jnp.dot(h2r, w2_ref[r],
                           preferred_element_type=jnp.float32)
        o_ref[...] = acc + b2_ref[...]

    return pl.pallas_call(
        body,
        grid=(GRID,),
        in_specs=[
            pl.BlockSpec((NQ, RQ, NB, E), lambda i: (0, 0, i, 0)),
            pl.BlockSpec((NB, NC * RP), lambda i: (i, 0)),
            pl.BlockSpec((1, RP), lambda i: (0, 0)),
            pl.BlockSpec((RP, E, C), lambda i: (0, 0, 0)),
            pl.BlockSpec((1, C), lambda i: (0, 0)),
        ],
        out_specs=pl.BlockSpec((NB, C), lambda i: (i, 0)),
        out_shape=jax.ShapeDtypeStruct((N, C), jnp.float32),
    )


def kernel(weights1, weights2, bias1, bias2, relation_embeddings, row_indices,
           col_indices, hor_indices, ver_indices, nt):
    RP, N, E = weights1.shape
    C = weights2.shape[2]
    T = row_indices.shape[0]
    nt_s = hor_indices.shape[0] // RP

    CH = T // NW
    WIN = CH + L
    SB = 2000
    TR = 2048
    NB = 1000

    fr = hor_indices[:nt_s, 0]
    to_ = hor_indices[:nt_s, 1]
    frp = jnp.pad(fr, (0, WIN + 8))
    top = jnp.pad(to_, (0, WIN + 8))
    rdiag = jnp.diagonal(relation_embeddings).astype(jnp.float32)
    w1f = weights1.reshape(RP * N, E)

    k1 = _make_k1(T, N, RP, nt_s + WIN + 8, CH, WIN)
    subj, obj, degkey, w1key, deg = k1(
        row_indices, col_indices, frp, top, rdiag)

    k1c = _make_k1c(N, RP)
    scale = k1c(deg.reshape(NC, RP, N), rdiag.reshape(RP, 1)).reshape(RP * N)

    k2 = _make_k2(T, N, RP, E, CH, SB)
    hpart = k2(subj, w1key, degkey, scale, w1f)

    k2b = _make_k2b(N, E, 128)
    btile = jnp.tile(bias1, 128 // E).reshape(1, 128)
    hrelu = k2b(hpart.reshape(NC, N * E // 128, 128), btile).reshape(N, E)

    k3 = _make_k3(T, N, RP, E, SB, TR, 256)
    s2 = k3(obj, degkey, hrelu)

    k4 = _make_k4(N, RP, E, C, NB)
    degt = jnp.transpose(deg.reshape(NC * RP, N))
    out = k4(s2.reshape(4, RP // 4, N, E),
             degt,
             rdiag.reshape(1, RP),
             weights2,
             bias2.reshape(1, C))
    return out


# speedup vs baseline: 1.0182x; 1.0182x over previous
import jax
import jax.numpy as jnp
from jax import lax
from jax.experimental import pallas as pl
from jax.experimental.pallas import tpu as pltpu
from jax.experimental.pallas import tpu_sc as plsc

NC = 2
NS = 16
L = 16
NW = NC * NS


def _mesh():
    return plsc.VectorSubcoreMesh(core_axis_name="c", subcore_axis_name="s")


def _make_k1(T, N, RP, NT_PAD, CH, WIN):
    NRP = RP * N
    ZSL = NRP // NS

    def body(rows_h, cols_h, fr_h, to_h, rdiag_h,
             subj_h, obj_h, degkey_h, w1key_h, deg_h,
             rows_v, cols_v, frw_v, tow_v, sv_v, ov_v, dk_v, wk_v, val_v,
             rdiag_v, zero_v, deg_s):
        cid = lax.axis_index("c")
        sid = lax.axis_index("s")
        wid = sid * NC + cid
        t0 = wid * CH
        pltpu.sync_copy(rows_h.at[pl.ds(t0, CH)], rows_v)
        pltpu.sync_copy(cols_h.at[pl.ds(t0, CH)], cols_v)
        pltpu.sync_copy(rdiag_h, rdiag_v)

        def _z(i, c):
            zero_v[pl.ds(i * L, L)] = jnp.zeros((L,), jnp.float32)
            return c
        lax.fori_loop(0, ZSL // L, _z, 0)
        pltpu.sync_copy(zero_v, deg_s.at[pl.ds(sid * ZSL, ZSL)])

        base = rows_v[pl.ds(0, L)][0]
        base_al = (base // 8) * 8
        pltpu.sync_copy(fr_h.at[pl.ds(base_al, WIN)], frw_v)
        pltpu.sync_copy(to_h.at[pl.ds(base_al, WIN)], tow_v)

        def _g(g, c):
            o = g * L
            idx = rows_v[pl.ds(o, L)] - base_al
            sv = plsc.load_gather(frw_v, [idx])
            ov = plsc.load_gather(tow_v, [idx])
            cv = cols_v[pl.ds(o, L)]
            vv = plsc.load_gather(rdiag_v, [cv])
            sv_v[0, pl.ds(o, L)] = sv
            ov_v[0, pl.ds(o, L)] = ov
            dk_v[0, pl.ds(o, L)] = cv * N + sv
            wk_v[0, pl.ds(o, L)] = cv * N + ov
            val_v[0, pl.ds(o, L)] = vv
            return c
        lax.fori_loop(0, CH // L, _g, 0)

        pltpu.sync_copy(sv_v.at[0], subj_h.at[pl.ds(t0, CH)])
        pltpu.sync_copy(ov_v.at[0], obj_h.at[pl.ds(t0, CH)])
        pltpu.sync_copy(dk_v.at[0], degkey_h.at[pl.ds(t0, CH)])
        pltpu.sync_copy(wk_v.at[0], w1key_h.at[pl.ds(t0, CH)])

        plsc.subcore_barrier()
        pltpu.sync_copy(val_v.at[0], deg_s.at[dk_v.at[0]], add=True)
        plsc.subcore_barrier()

        @pl.when(sid == 0)
        def _():
            pltpu.sync_copy(deg_s, deg_h.at[cid])

    i32, f32 = jnp.int32, jnp.float32
    return pl.kernel(
        body,
        out_type=(
            jax.ShapeDtypeStruct((T,), i32),
            jax.ShapeDtypeStruct((T,), i32),
            jax.ShapeDtypeStruct((T,), i32),
            jax.ShapeDtypeStruct((T,), i32),
            jax.ShapeDtypeStruct((NC, NRP), f32),
        ),
        mesh=_mesh(),
        compiler_params=pltpu.CompilerParams(needs_layout_passes=False, use_tc_tiling_on_sc=False),
        scratch_types=[
            pltpu.VMEM((CH,), i32),
            pltpu.VMEM((CH,), i32),
            pltpu.VMEM((WIN,), i32),
            pltpu.VMEM((WIN,), i32),
            pltpu.VMEM((1, CH), i32),
            pltpu.VMEM((1, CH), i32),
            pltpu.VMEM((1, CH), i32),
            pltpu.VMEM((1, CH), i32),
            pltpu.VMEM((1, CH), f32),
            pltpu.VMEM((L,), f32),
            pltpu.VMEM((ZSL,), f32),
            pltpu.VMEM_SHARED((RP * N,), f32),
        ],
    )


def _make_k1c(N, RP):
    def body(deg_ref, rd_ref, o_ref):
        d = deg_ref[0] + deg_ref[1]
        o_ref[...] = jnp.where(d > 0.0, rd_ref[...] / d, 0.0)

    return pl.pallas_call(
        body,
        in_specs=[
            pl.BlockSpec((NC, RP, N), lambda: (0, 0, 0)),
            pl.BlockSpec((RP, 1), lambda: (0, 0)),
        ],
        out_specs=pl.BlockSpec((RP, N), lambda: (0, 0)),
        out_shape=jax.ShapeDtypeStruct((RP, N), jnp.float32),
    )


def _make_k2(T, N, RP, E, CH, SB):
    NSUB = CH // SB
    RPS = N // NS

    def body(subj_h, w1key_h, degkey_h, scale_h, w1_h,
             hpart_h,
             sv0, sv1, wk0, wk1, dk0, dk1, sc0, sc1, rw0, rw1, zero_v, h_s,
             semr0, semr1, sems0, sems1):
        cid = lax.axis_index("c")
        sid = lax.axis_index("s")
        wid = sid * NC + cid
        t0 = wid * CH
        iota = lax.iota(jnp.int32, L)
        sv = [sv0, sv1]
        wk = [wk0, wk1]
        dk = [dk0, dk1]
        scb = [sc0, sc1]
        rw = [rw0, rw1]
        semr = [semr0, semr1]
        sems = [sems0, sems1]

        def _z(i, c):
            zero_v[i, :] = jnp.zeros((L,), jnp.float32)
            return c
        lax.fori_loop(0, RPS, _z, 0)
        pltpu.sync_copy(zero_v, h_s.at[pl.ds(sid * RPS, RPS)])
        plsc.subcore_barrier()

        def _issue(i, b):
            off = t0 + i * SB
            pltpu.sync_copy(w1key_h.at[pl.ds(off, SB)], wk[b].at[0])
            pltpu.sync_copy(degkey_h.at[pl.ds(off, SB)], dk[b].at[0])
            pltpu.sync_copy(subj_h.at[pl.ds(off, SB)], sv[b].at[0])
            return (pltpu.async_copy(w1_h.at[wk[b].at[0]], rw[b], semr[b]),
                    pltpu.async_copy(scale_h.at[dk[b].at[0]], scb[b], sems[b]))

        pend = {0: _issue(0, 0)}
        for sc in range(NSUB):
            b = sc % 2
            for cp in pend.pop(sc):
                cp.wait()
            if sc + 1 < NSUB:
                pend[sc + 1] = _issue(sc + 1, (sc + 1) % 2)

            def _m(g, c):
                o = g * L
                s = scb[b][pl.ds(o, L)]
                ridx = o + iota
                for j in range(E):
                    jv = jnp.full((L,), j, dtype=jnp.int32)
                    col = plsc.load_gather(rw[b], [ridx, jv])
                    plsc.store_scatter(rw[b], [ridx, jv], col * s)
                return c
            lax.fori_loop(0, SB // L, _m, 0)
            pltpu.sync_copy(rw[b], h_s.at[sv[b].at[0]], add=True)

        plsc.subcore_barrier()

        @pl.when(sid == 0)
        def _():
            pltpu.sync_copy(h_s, hpart_h.at[cid])

    i32, f32 = jnp.int32, jnp.float32
    return pl.kernel(
        body,
        out_type=jax.ShapeDtypeStruct((NC, N, E), f32),
        mesh=_mesh(),
        compiler_params=pltpu.CompilerParams(needs_layout_passes=False, use_tc_tiling_on_sc=False),
        scratch_types=[
            pltpu.VMEM((1, SB), i32),
            pltpu.VMEM((1, SB), i32),
            pltpu.VMEM((1, SB), i32),
            pltpu.VMEM((1, SB), i32),
            pltpu.VMEM((1, SB), i32),
            pltpu.VMEM((1, SB), i32),
            pltpu.VMEM((SB,), f32),
            pltpu.VMEM((SB,), f32),
            pltpu.VMEM((SB, E), f32),
            pltpu.VMEM((SB, E), f32),
            pltpu.VMEM((RPS, E), f32),
            pltpu.VMEM_SHARED((N, E), f32),
            pltpu.SemaphoreType.DMA,
            pltpu.SemaphoreType.DMA,
            pltpu.SemaphoreType.DMA,
            pltpu.SemaphoreType.DMA,
        ],
    )


def _make_k2b(N, E, NCOLS):
    NR = N * E // NCOLS

    def body(p_ref, b_ref, o_ref):
        o_ref[...] = jnp.maximum(p_ref[0] + p_ref[1] + b_ref[...], 0.0)

    return pl.pallas_call(
        body,
        out_shape=jax.ShapeDtypeStruct((NR, NCOLS), jnp.float32),
    )


def _make_k3(T, N, RP, E, SB, TR, BLK):
    NQ = 4
    QN = (RP // NQ) * N
    CH3 = T // NS
    NSUB = CH3 // SB
    ZR = (QN + TR) // NS
    ZB = ZR // 4
    CSB = SB + BLK + 2 * L

    def body(obj_h, degkey_h, hrelu_h,
             s2_h,
             ob0, ob1, dk0, dk1, cob_v, clk_v, blk_v, hrows_v, zero_v, s2_s,
             semg, semo0, semo1, semd0, semd1):
        cid = lax.axis_index("c")
        sid = lax.axis_index("s")
        iota = lax.iota(jnp.int32, L)
        ob = [ob0, ob1]
        dk = [dk0, dk1]
        semo = [semo0, semo1]
        semd = [semd0, semd1]

        for p in range(NQ // NC):
            q = NC * p + cid
            rbase = q * QN

            def _z(i, c):
                zero_v[i, :] = jnp.zeros((L,), jnp.float32)
                return c
            lax.fori_loop(0, ZB, _z, 0)
            for z in range(4):
                pltpu.sync_copy(zero_v, s2_s.at[pl.ds(sid * ZR + z * ZB, ZB)])
            plsc.subcore_barrier()

            def _issue(i, b):
                off = sid * CH3 + i * SB
                return (
                    pltpu.async_copy(obj_h.at[pl.ds(off, SB)], ob[b].at[0],
                                     semo[b]),
                    pltpu.async_copy(degkey_h.at[pl.ds(off, SB)], dk[b].at[0],
                                     semd[b]),
                )

            pend = {0: _issue(0, 0)}
            for sc in range(NSUB):
                b = sc % 2
                for cp in pend.pop(sc):
                    cp.wait()
                if sc + 1 < NSUB:
                    pend[sc + 1] = _issue(sc + 1, (sc + 1) % 2)

                def _cg(g, cnt):
                    o = g * L
                    kk = dk[b][0, pl.ds(o, L)] - rbase
                    m = (kk >= 0) & (kk < QN)
                    ov = ob[b][0, pl.ds(o, L)]
                    plsc.store_compressed(cob_v.at[pl.ds(cnt, L)], ov, mask=m)
                    plsc.store_compressed(clk_v.at[pl.ds(cnt, L)], kk, mask=m)
                    return cnt + plsc.all_reduce_population_count(m)[0]
                cnt = lax.fori_loop(0, SB // L, _cg, 0)

                def _pad(i, c):
                    o2 = cnt + i * L
                    cob_v[pl.ds(o2, L)] = (o2 + iota) & (L - 1)
                    clk_v[pl.ds(o2, L)] = QN + ((o2 + iota) & (TR - 1))
                    return c
                lax.fori_loop(0, BLK // L + 1, _pad, 0)

                nblk = (cnt + BLK - 1) // BLK

                @pl.when(nblk > 0)
                def _():
                    pltpu.async_copy(hrelu_h.at[cob_v.at[pl.ds(0, BLK)]],
                                     hrows_v.at[0], semg)

                def _bl(b2, c):
                    cur = b2 % 2

                    @pl.when(b2 + 1 < nblk)
                    def _():
                        o3 = (b2 + 1) * BLK
                        pltpu.async_copy(
                            hrelu_h.at[cob_v.at[pl.ds(o3, BLK)]],
                            hrows_v.at[(b2 + 1) % 2], semg)

                    pltpu.make_async_copy(hrelu_h.at[pl.ds(0, BLK)],
                                          hrows_v.at[0], semg).wait()
                    o2 = b2 * BLK

                    def _cp(i, c2):
                        blk_v[0, pl.ds(i * L, L)] = clk_v[pl.ds(o2 + i * L, L)]
                        return c2
                    lax.fori_loop(0, BLK // L, _cp, 0)
                    pltpu.sync_copy(hrows_v.at[cur], s2_s.at[blk_v.at[0]],
                                    add=True)
                    return c
                lax.fori_loop(0, nblk, _bl, 0)

            plsc.subcore_barrier()

            @pl.when(sid == 0)
            def _():
                pltpu.sync_copy(s2_s.at[pl.ds(0, QN)], s2_h.at[q])
            plsc.subcore_barrier()

    i32, f32 = jnp.int32, jnp.float32
    return pl.kernel(
        body,
        out_type=jax.ShapeDtypeStruct((NQ, QN, E), f32),
        mesh=_mesh(),
        compiler_params=pltpu.CompilerParams(needs_layout_passes=False, use_tc_tiling_on_sc=False),
        scratch_types=[
            pltpu.VMEM((1, SB), i32),
            pltpu.VMEM((1, SB), i32),
            pltpu.VMEM((1, SB), i32),
            pltpu.VMEM((1, SB), i32),
            pltpu.VMEM((CSB,), i32),
            pltpu.VMEM((CSB,), i32),
            pltpu.VMEM((1, BLK), i32),
            pltpu.VMEM((2, BLK, E), f32),
            pltpu.VMEM((ZB, E), f32),
            pltpu.VMEM_SHARED((QN + TR, E), f32),
            pltpu.SemaphoreType.DMA,
            pltpu.SemaphoreType.DMA,
            pltpu.SemaphoreType.DMA,
            pltpu.SemaphoreType.DMA,
            pltpu.SemaphoreType.DMA,
        ],
    )


def _make_k4(N, RP, E, C, NB):
    NQ = 4
    RQ = RP // NQ
    GRID = N // NB

    def body(s2_ref, deg_ref, rd_ref, w2_ref, b2_ref, o_ref):
        d = deg_ref[:, :RP] + deg_ref[:, RP:]
        scale = jnp.where(d > 0.0, rd_ref[...] / d, 0.0)
        acc = jnp.zeros((NB, C), jnp.float32)
        for r in range(RP):
            h2r = s2_ref[r // RQ, r % RQ] * scale[:, r][:, None]
            acc += jnp.dot(h2r, w2_ref[r],
                           preferred_element_type=jnp.float32)
        o_ref[...] = acc + b2_ref[...]

    return pl.pallas_call(
        body,
        grid=(GRID,),
        in_specs=[
            pl.BlockSpec((NQ, RQ, NB, E), lambda i: (0, 0, i, 0)),
            pl.BlockSpec((NB, NC * RP), lambda i: (i, 0)),
            pl.BlockSpec((1, RP), lambda i: (0, 0)),
            pl.BlockSpec((RP, E, C), lambda i: (0, 0, 0)),
            pl.BlockSpec((1, C), lambda i: (0, 0)),
        ],
        out_specs=pl.BlockSpec((NB, C), lambda i: (i, 0)),
        out_shape=jax.ShapeDtypeStruct((N, C), jnp.float32),
    )


def kernel(weights1, weights2, bias1, bias2, relation_embeddings, row_indices,
           col_indices, hor_indices, ver_indices, nt):
    RP, N, E = weights1.shape
    C = weights2.shape[2]
    T = row_indices.shape[0]
    nt_s = hor_indices.shape[0] // RP

    CH = T // NW
    WIN = CH + L
    SB = 2000
    TR = 2048
    NB = 1000

    fr = hor_indices[:nt_s, 0]
    to_ = hor_indices[:nt_s, 1]
    frp = jnp.pad(fr, (0, WIN + 8))
    top = jnp.pad(to_, (0, WIN + 8))
    rdiag = jnp.diagonal(relation_embeddings).astype(jnp.float32)
    w1f = weights1.reshape(RP * N, E)

    k1 = _make_k1(T, N, RP, nt_s + WIN + 8, CH, WIN)
    subj, obj, degkey, w1key, deg = k1(
        row_indices, col_indices, frp, top, rdiag)

    k1c = _make_k1c(N, RP)
    scale = k1c(deg.reshape(NC, RP, N), rdiag.reshape(RP, 1)).reshape(RP * N)

    k2 = _make_k2(T, N, RP, E, CH, SB)
    hpart = k2(subj, w1key, degkey, scale, w1f)

    k2b = _make_k2b(N, E, 128)
    btile = jnp.tile(bias1, 128 // E).reshape(1, 128)
    hrelu = k2b(hpart.reshape(NC, N * E // 128, 128), btile).reshape(N, E)

    k3 = _make_k3(T, N, RP, E, SB, TR, 256)
    s2 = k3(obj, degkey, hrelu)

    k4 = _make_k4(N, RP, E, C, NB)
    degt = jnp.transpose(deg.reshape(NC * RP, N))
    out = k4(s2.reshape(4, RP // 4, N, E),
             degt,
             rdiag.reshape(1, RP),
             weights2,
             bias2.reshape(1, C))
    return out


# speedup vs baseline: 1.1942x; 1.1728x over previous
import jax
import jax.numpy as jnp
from jax import lax
from jax.experimental import pallas as pl
from jax.experimental.pallas import tpu as pltpu
from jax.experimental.pallas import tpu_sc as plsc

NC = 2
NS = 16
L = 16
NW = NC * NS


def _mesh():
    return plsc.VectorSubcoreMesh(core_axis_name="c", subcore_axis_name="s")


def _make_k1(T, N, RP, NT_PAD, CH, WIN):
    NRP = RP * N
    ZSL = NRP // NS

    def body(rows_h, cols_h, fr_h, to_h, rdiag_h,
             subj_h, obj_h, degkey_h, w1key_h, deg_h,
             rows_v, cols_v, frw_v, tow_v, sv_v, ov_v, dk_v, wk_v, val_v,
             rdiag_v, zero_v, deg_s):
        cid = lax.axis_index("c")
        sid = lax.axis_index("s")
        wid = sid * NC + cid
        t0 = wid * CH
        pltpu.sync_copy(rows_h.at[pl.ds(t0, CH)], rows_v)
        pltpu.sync_copy(cols_h.at[pl.ds(t0, CH)], cols_v)
        pltpu.sync_copy(rdiag_h, rdiag_v)

        def _z(i, c):
            zero_v[pl.ds(i * L, L)] = jnp.zeros((L,), jnp.float32)
            return c
        lax.fori_loop(0, ZSL // L, _z, 0)
        pltpu.sync_copy(zero_v, deg_s.at[pl.ds(sid * ZSL, ZSL)])

        base = rows_v[pl.ds(0, L)][0]
        base_al = (base // 8) * 8
        pltpu.sync_copy(fr_h.at[pl.ds(base_al, WIN)], frw_v)
        pltpu.sync_copy(to_h.at[pl.ds(base_al, WIN)], tow_v)

        def _g(g, c):
            o = g * L
            idx = rows_v[pl.ds(o, L)] - base_al
            sv = plsc.load_gather(frw_v, [idx])
            ov = plsc.load_gather(tow_v, [idx])
            cv = cols_v[pl.ds(o, L)]
            vv = plsc.load_gather(rdiag_v, [cv])
            sv_v[0, pl.ds(o, L)] = sv
            ov_v[0, pl.ds(o, L)] = ov
            dk_v[0, pl.ds(o, L)] = cv * N + sv
            wk_v[0, pl.ds(o, L)] = cv * N + ov
            val_v[0, pl.ds(o, L)] = vv
            return c
        lax.fori_loop(0, CH // L, _g, 0)

        pltpu.sync_copy(sv_v.at[0], subj_h.at[pl.ds(t0, CH)])
        pltpu.sync_copy(ov_v.at[0], obj_h.at[pl.ds(t0, CH)])
        pltpu.sync_copy(dk_v.at[0], degkey_h.at[pl.ds(t0, CH)])
        pltpu.sync_copy(wk_v.at[0], w1key_h.at[pl.ds(t0, CH)])

        plsc.subcore_barrier()
        pltpu.sync_copy(val_v.at[0], deg_s.at[dk_v.at[0]], add=True)
        plsc.subcore_barrier()

        @pl.when(sid == 0)
        def _():
            pltpu.sync_copy(deg_s, deg_h.at[cid])

    i32, f32 = jnp.int32, jnp.float32
    return pl.kernel(
        body,
        out_type=(
            jax.ShapeDtypeStruct((T,), i32),
            jax.ShapeDtypeStruct((T,), i32),
            jax.ShapeDtypeStruct((T,), i32),
            jax.ShapeDtypeStruct((T,), i32),
            jax.ShapeDtypeStruct((NC, NRP), f32),
        ),
        mesh=_mesh(),
        compiler_params=pltpu.CompilerParams(needs_layout_passes=False, use_tc_tiling_on_sc=False),
        scratch_types=[
            pltpu.VMEM((CH,), i32),
            pltpu.VMEM((CH,), i32),
            pltpu.VMEM((WIN,), i32),
            pltpu.VMEM((WIN,), i32),
            pltpu.VMEM((1, CH), i32),
            pltpu.VMEM((1, CH), i32),
            pltpu.VMEM((1, CH), i32),
            pltpu.VMEM((1, CH), i32),
            pltpu.VMEM((1, CH), f32),
            pltpu.VMEM((L,), f32),
            pltpu.VMEM((ZSL,), f32),
            pltpu.VMEM_SHARED((RP * N,), f32),
        ],
    )


def _make_k1c(N, RP):
    def body(deg_ref, rd_ref, o_ref):
        d = deg_ref[0] + deg_ref[1]
        o_ref[...] = jnp.where(d > 0.0, rd_ref[...] / d, 0.0)

    return pl.pallas_call(
        body,
        in_specs=[
            pl.BlockSpec((NC, RP, N), lambda: (0, 0, 0)),
            pl.BlockSpec((RP, 1), lambda: (0, 0)),
        ],
        out_specs=pl.BlockSpec((RP, N), lambda: (0, 0)),
        out_shape=jax.ShapeDtypeStruct((RP, N), jnp.float32),
    )


def _make_k2(T, N, RP, E, CH, SB):
    NSUB = CH // SB
    RPS = N // NS

    def body(subj_h, w1key_h, degkey_h, scale_h, w1_h,
             hpart_h,
             sv0, sv1, wk0, wk1, dk0, dk1, sc0, sc1, rw0, rw1, zero_v, h_s,
             semr0, semr1, sems0, sems1):
        cid = lax.axis_index("c")
        sid = lax.axis_index("s")
        wid = sid * NC + cid
        t0 = wid * CH
        iota = lax.iota(jnp.int32, L)
        sv = [sv0, sv1]
        wk = [wk0, wk1]
        dk = [dk0, dk1]
        scb = [sc0, sc1]
        rw = [rw0, rw1]
        semr = [semr0, semr1]
        sems = [sems0, sems1]

        def _z(i, c):
            zero_v[i, :] = jnp.zeros((L,), jnp.float32)
            return c
        lax.fori_loop(0, RPS, _z, 0)
        pltpu.sync_copy(zero_v, h_s.at[pl.ds(sid * RPS, RPS)])
        plsc.subcore_barrier()

        def _issue(i, b):
            off = t0 + i * SB
            pltpu.sync_copy(w1key_h.at[pl.ds(off, SB)], wk[b].at[0])
            pltpu.sync_copy(degkey_h.at[pl.ds(off, SB)], dk[b].at[0])
            pltpu.sync_copy(subj_h.at[pl.ds(off, SB)], sv[b].at[0])
            return (pltpu.async_copy(w1_h.at[wk[b].at[0]], rw[b], semr[b]),
                    pltpu.async_copy(scale_h.at[dk[b].at[0]], scb[b], sems[b]))

        pend = {0: _issue(0, 0)}
        for sc in range(NSUB):
            b = sc % 2
            for cp in pend.pop(sc):
                cp.wait()
            if sc + 1 < NSUB:
                pend[sc + 1] = _issue(sc + 1, (sc + 1) % 2)

            def _m(g, c):
                o = g * L
                s = scb[b][pl.ds(o, L)]
                ridx = o + iota
                for j in range(E):
                    jv = jnp.full((L,), j, dtype=jnp.int32)
                    col = plsc.load_gather(rw[b], [ridx, jv])
                    plsc.store_scatter(rw[b], [ridx, jv], col * s)
                return c
            lax.fori_loop(0, SB // L, _m, 0)
            pltpu.sync_copy(rw[b], h_s.at[sv[b].at[0]], add=True)

        plsc.subcore_barrier()

        @pl.when(sid == 0)
        def _():
            pltpu.sync_copy(h_s, hpart_h.at[cid])

    i32, f32 = jnp.int32, jnp.float32
    return pl.kernel(
        body,
        out_type=jax.ShapeDtypeStruct((NC, N, E), f32),
        mesh=_mesh(),
        compiler_params=pltpu.CompilerParams(needs_layout_passes=False, use_tc_tiling_on_sc=False),
        scratch_types=[
            pltpu.VMEM((1, SB), i32),
            pltpu.VMEM((1, SB), i32),
            pltpu.VMEM((1, SB), i32),
            pltpu.VMEM((1, SB), i32),
            pltpu.VMEM((1, SB), i32),
            pltpu.VMEM((1, SB), i32),
            pltpu.VMEM((SB,), f32),
            pltpu.VMEM((SB,), f32),
            pltpu.VMEM((SB, E), f32),
            pltpu.VMEM((SB, E), f32),
            pltpu.VMEM((RPS, E), f32),
            pltpu.VMEM_SHARED((N, E), f32),
            pltpu.SemaphoreType.DMA,
            pltpu.SemaphoreType.DMA,
            pltpu.SemaphoreType.DMA,
            pltpu.SemaphoreType.DMA,
        ],
    )


def _make_k2b(N, E, NCOLS):
    NR = N * E // NCOLS

    def body(p_ref, b_ref, o_ref):
        o_ref[...] = jnp.maximum(p_ref[0] + p_ref[1] + b_ref[...], 0.0)

    return pl.pallas_call(
        body,
        out_shape=jax.ShapeDtypeStruct((NR, NCOLS), jnp.float32),
    )


def _make_k3(T, N, RP, E, SB, TR, BLK):
    NQ = 4
    QN = (RP // NQ) * N
    CH3 = T // NS
    NSUB = CH3 // SB
    ZR = (QN + TR) // NS
    ZB = ZR // 4
    CSB = SB + BLK + 2 * L

    def body(obj_h, degkey_h, hrelu_h,
             s2_h,
             ob0, ob1, dk0, dk1, blk_v, hrows_v, zero_v, s2_s,
             semo0, semo1):
        cid = lax.axis_index("c")
        sid = lax.axis_index("s")
        iota = lax.iota(jnp.int32, L)
        ob = [ob0, ob1]
        dk = [dk0, dk1]
        semo = [semo0, semo1]

        for p in range(NQ // NC):
            q = NC * p + cid
            rbase = q * QN

            def _z(i, c):
                zero_v[i, :] = jnp.zeros((L,), jnp.float32)
                return c
            lax.fori_loop(0, ZB, _z, 0)
            for z in range(4):
                pltpu.sync_copy(zero_v, s2_s.at[pl.ds(sid * ZR + z * ZB, ZB)])
            plsc.subcore_barrier()

            def _issue(i, b):
                off = sid * CH3 + i * SB
                pltpu.sync_copy(obj_h.at[pl.ds(off, SB)], ob[b].at[0])
                pltpu.sync_copy(degkey_h.at[pl.ds(off, SB)], dk[b].at[0])
                return pltpu.async_copy(hrelu_h.at[ob[b].at[0]],
                                        hrows_v.at[b], semo[b])

            pend = {0: _issue(0, 0)}
            for sc in range(NSUB):
                b = sc % 2
                pend.pop(sc).wait()
                if sc + 1 < NSUB:
                    pend[sc + 1] = _issue(sc + 1, (sc + 1) % 2)

                def _f(g, c):
                    o = g * L
                    kk = dk[b][0, pl.ds(o, L)] - rbase
                    m = (kk >= 0) & (kk < QN)
                    tr = QN + ((o + iota) & (TR - 1))
                    blk_v[0, pl.ds(o, L)] = jnp.where(m, kk, tr)
                    return c
                lax.fori_loop(0, SB // L, _f, 0)
                pltpu.sync_copy(hrows_v.at[b], s2_s.at[blk_v.at[0]],
                                add=True)

            plsc.subcore_barrier()

            @pl.when(sid == 0)
            def _():
                pltpu.sync_copy(s2_s.at[pl.ds(0, QN)], s2_h.at[q])
            plsc.subcore_barrier()

    i32, f32 = jnp.int32, jnp.float32
    return pl.kernel(
        body,
        out_type=jax.ShapeDtypeStruct((NQ, QN, E), f32),
        mesh=_mesh(),
        compiler_params=pltpu.CompilerParams(needs_layout_passes=False, use_tc_tiling_on_sc=False),
        scratch_types=[
            pltpu.VMEM((1, SB), i32),
            pltpu.VMEM((1, SB), i32),
            pltpu.VMEM((1, SB), i32),
            pltpu.VMEM((1, SB), i32),
            pltpu.VMEM((1, SB), i32),
            pltpu.VMEM((2, SB, E), f32),
            pltpu.VMEM((ZB, E), f32),
            pltpu.VMEM_SHARED((QN + TR, E), f32),
            pltpu.SemaphoreType.DMA,
            pltpu.SemaphoreType.DMA,
        ],
    )


def _make_k4(N, RP, E, C, NB):
    NQ = 4
    RQ = RP // NQ
    GRID = N // NB

    def body(s2_ref, deg_ref, rd_ref, w2_ref, b2_ref, o_ref):
        d = deg_ref[:, :RP] + deg_ref[:, RP:]
        scale = jnp.where(d > 0.0, rd_ref[...] / d, 0.0)
        acc = jnp.zeros((NB, C), jnp.float32)
        for r in range(RP):
            h2r = s2_ref[r // RQ, r % RQ] * scale[:, r][:, None]
            acc += jnp.dot(h2r, w2_ref[r],
                           preferred_element_type=jnp.float32)
        o_ref[...] = acc + b2_ref[...]

    return pl.pallas_call(
        body,
        grid=(GRID,),
        in_specs=[
            pl.BlockSpec((NQ, RQ, NB, E), lambda i: (0, 0, i, 0)),
            pl.BlockSpec((NB, NC * RP), lambda i: (i, 0)),
            pl.BlockSpec((1, RP), lambda i: (0, 0)),
            pl.BlockSpec((RP, E, C), lambda i: (0, 0, 0)),
            pl.BlockSpec((1, C), lambda i: (0, 0)),
        ],
        out_specs=pl.BlockSpec((NB, C), lambda i: (i, 0)),
        out_shape=jax.ShapeDtypeStruct((N, C), jnp.float32),
    )


def kernel(weights1, weights2, bias1, bias2, relation_embeddings, row_indices,
           col_indices, hor_indices, ver_indices, nt):
    RP, N, E = weights1.shape
    C = weights2.shape[2]
    T = row_indices.shape[0]
    nt_s = hor_indices.shape[0] // RP

    CH = T // NW
    WIN = CH + L
    SB = 2000
    TR = 2048
    NB = 1000

    fr = hor_indices[:nt_s, 0]
    to_ = hor_indices[:nt_s, 1]
    frp = jnp.pad(fr, (0, WIN + 8))
    top = jnp.pad(to_, (0, WIN + 8))
    rdiag = jnp.diagonal(relation_embeddings).astype(jnp.float32)
    w1f = weights1.reshape(RP * N, E)

    k1 = _make_k1(T, N, RP, nt_s + WIN + 8, CH, WIN)
    subj, obj, degkey, w1key, deg = k1(
        row_indices, col_indices, frp, top, rdiag)

    k1c = _make_k1c(N, RP)
    scale = k1c(deg.reshape(NC, RP, N), rdiag.reshape(RP, 1)).reshape(RP * N)

    k2 = _make_k2(T, N, RP, E, CH, SB)
    hpart = k2(subj, w1key, degkey, scale, w1f)

    k2b = _make_k2b(N, E, 128)
    btile = jnp.tile(bias1, 128 // E).reshape(1, 128)
    hrelu = k2b(hpart.reshape(NC, N * E // 128, 128), btile).reshape(N, E)

    k3 = _make_k3(T, N, RP, E, SB, TR, 256)
    s2 = k3(obj, degkey, hrelu)

    k4 = _make_k4(N, RP, E, C, NB)
    degt = jnp.transpose(deg.reshape(NC * RP, N))
    out = k4(s2.reshape(4, RP // 4, N, E),
             degt,
             rdiag.reshape(1, RP),
             weights2,
             bias2.reshape(1, C))
    return out


# speedup vs baseline: 1.1977x; 1.0029x over previous
import jax
import jax.numpy as jnp
from jax import lax
from jax.experimental import pallas as pl
from jax.experimental.pallas import tpu as pltpu
from jax.experimental.pallas import tpu_sc as plsc

NC = 2
NS = 16
L = 16
NW = NC * NS


def _mesh():
    return plsc.VectorSubcoreMesh(core_axis_name="c", subcore_axis_name="s")


def _make_k1(T, N, RP, NT_PAD, CH, WIN):
    NRP = RP * N
    ZSL = NRP // NS

    def body(rows_h, cols_h, fr_h, to_h, rdiag_h,
             subj_h, obj_h, degkey_h, w1key_h, deg_h,
             rows_v, cols_v, frw_v, tow_v, sv_v, ov_v, dk_v, wk_v, val_v,
             rdiag_v, zero_v, deg_s):
        cid = lax.axis_index("c")
        sid = lax.axis_index("s")
        wid = sid * NC + cid
        t0 = wid * CH
        pltpu.sync_copy(rows_h.at[pl.ds(t0, CH)], rows_v)
        pltpu.sync_copy(cols_h.at[pl.ds(t0, CH)], cols_v)
        pltpu.sync_copy(rdiag_h, rdiag_v)

        def _z(i, c):
            zero_v[pl.ds(i * L, L)] = jnp.zeros((L,), jnp.float32)
            return c
        lax.fori_loop(0, ZSL // L, _z, 0)
        pltpu.sync_copy(zero_v, deg_s.at[pl.ds(sid * ZSL, ZSL)])

        base = rows_v[pl.ds(0, L)][0]
        base_al = (base // 8) * 8
        pltpu.sync_copy(fr_h.at[pl.ds(base_al, WIN)], frw_v)
        pltpu.sync_copy(to_h.at[pl.ds(base_al, WIN)], tow_v)

        def _g(g, c):
            o = g * L
            idx = rows_v[pl.ds(o, L)] - base_al
            sv = plsc.load_gather(frw_v, [idx])
            ov = plsc.load_gather(tow_v, [idx])
            cv = cols_v[pl.ds(o, L)]
            vv = plsc.load_gather(rdiag_v, [cv])
            sv_v[0, pl.ds(o, L)] = sv
            ov_v[0, pl.ds(o, L)] = ov
            dk_v[0, pl.ds(o, L)] = cv * N + sv
            wk_v[0, pl.ds(o, L)] = cv * N + ov
            val_v[0, pl.ds(o, L)] = vv
            return c
        lax.fori_loop(0, CH // L, _g, 0)

        pltpu.sync_copy(sv_v.at[0], subj_h.at[pl.ds(t0, CH)])
        pltpu.sync_copy(ov_v.at[0], obj_h.at[pl.ds(t0, CH)])
        pltpu.sync_copy(dk_v.at[0], degkey_h.at[pl.ds(t0, CH)])
        pltpu.sync_copy(wk_v.at[0], w1key_h.at[pl.ds(t0, CH)])

        plsc.subcore_barrier()
        pltpu.sync_copy(val_v.at[0], deg_s.at[dk_v.at[0]], add=True)
        plsc.subcore_barrier()

        @pl.when(sid == 0)
        def _():
            pltpu.sync_copy(deg_s, deg_h.at[cid])

    i32, f32 = jnp.int32, jnp.float32
    return pl.kernel(
        body,
        out_type=(
            jax.ShapeDtypeStruct((T,), i32),
            jax.ShapeDtypeStruct((T,), i32),
            jax.ShapeDtypeStruct((T,), i32),
            jax.ShapeDtypeStruct((T,), i32),
            jax.ShapeDtypeStruct((NC, NRP), f32),
        ),
        mesh=_mesh(),
        compiler_params=pltpu.CompilerParams(needs_layout_passes=False, use_tc_tiling_on_sc=False),
        scratch_types=[
            pltpu.VMEM((CH,), i32),
            pltpu.VMEM((CH,), i32),
            pltpu.VMEM((WIN,), i32),
            pltpu.VMEM((WIN,), i32),
            pltpu.VMEM((1, CH), i32),
            pltpu.VMEM((1, CH), i32),
            pltpu.VMEM((1, CH), i32),
            pltpu.VMEM((1, CH), i32),
            pltpu.VMEM((1, CH), f32),
            pltpu.VMEM((L,), f32),
            pltpu.VMEM((ZSL,), f32),
            pltpu.VMEM_SHARED((RP * N,), f32),
        ],
    )


def _make_k1c(N, RP):
    def body(deg_ref, rd_ref, o_ref):
        d = deg_ref[0] + deg_ref[1]
        o_ref[...] = jnp.where(d > 0.0, rd_ref[...] / d, 0.0)

    return pl.pallas_call(
        body,
        in_specs=[
            pl.BlockSpec((NC, RP, N), lambda: (0, 0, 0)),
            pl.BlockSpec((RP, 1), lambda: (0, 0)),
        ],
        out_specs=pl.BlockSpec((RP, N), lambda: (0, 0)),
        out_shape=jax.ShapeDtypeStruct((RP, N), jnp.float32),
    )


def _make_k2(T, N, RP, E, CH, SB):
    NSUB = CH // SB
    RPS = N // NS

    def body(subj_h, w1key_h, degkey_h, scale_h, w1_h,
             hpart_h,
             sv0, sv1, wk0, wk1, dk0, dk1, sc0, sc1, rw0, rw1, zero_v, h_s,
             semr0, semr1, sems0, sems1, semw0, semw1):
        cid = lax.axis_index("c")
        sid = lax.axis_index("s")
        wid = sid * NC + cid
        t0 = wid * CH
        iota = lax.iota(jnp.int32, L)
        sv = [sv0, sv1]
        wk = [wk0, wk1]
        dk = [dk0, dk1]
        scb = [sc0, sc1]
        rw = [rw0, rw1]
        semr = [semr0, semr1]
        sems = [sems0, sems1]
        semw = [semw0, semw1]

        def _z(i, c):
            zero_v[i, :] = jnp.zeros((L,), jnp.float32)
            return c
        lax.fori_loop(0, RPS, _z, 0)
        pltpu.sync_copy(zero_v, h_s.at[pl.ds(sid * RPS, RPS)])
        plsc.subcore_barrier()

        def _issue(i, b):
            off = t0 + i * SB
            pltpu.sync_copy(w1key_h.at[pl.ds(off, SB)], wk[b].at[0])
            pltpu.sync_copy(degkey_h.at[pl.ds(off, SB)], dk[b].at[0])
            pltpu.sync_copy(subj_h.at[pl.ds(off, SB)], sv[b].at[0])
            return (pltpu.async_copy(w1_h.at[wk[b].at[0]], rw[b], semr[b]),
                    pltpu.async_copy(scale_h.at[dk[b].at[0]], scb[b], sems[b]))

        pend = {0: _issue(0, 0)}
        wpend = {}
        for sc in range(NSUB):
            b = sc % 2
            for cp in pend.pop(sc):
                cp.wait()
            if sc + 1 < NSUB:
                if sc - 1 in wpend:
                    wpend.pop(sc - 1).wait()
                pend[sc + 1] = _issue(sc + 1, (sc + 1) % 2)

            def _m(g, c):
                o = g * L
                s = scb[b][pl.ds(o, L)]
                ridx = o + iota
                for j in range(E):
                    jv = jnp.full((L,), j, dtype=jnp.int32)
                    col = plsc.load_gather(rw[b], [ridx, jv])
                    plsc.store_scatter(rw[b], [ridx, jv], col * s)
                return c
            lax.fori_loop(0, SB // L, _m, 0)
            wpend[sc] = pltpu.async_copy(rw[b], h_s.at[sv[b].at[0]],
                                         semw[b], add=True)
        for cp in wpend.values():
            cp.wait()

        plsc.subcore_barrier()

        @pl.when(sid == 0)
        def _():
            pltpu.sync_copy(h_s, hpart_h.at[cid])

    i32, f32 = jnp.int32, jnp.float32
    return pl.kernel(
        body,
        out_type=jax.ShapeDtypeStruct((NC, N, E), f32),
        mesh=_mesh(),
        compiler_params=pltpu.CompilerParams(needs_layout_passes=False, use_tc_tiling_on_sc=False),
        scratch_types=[
            pltpu.VMEM((1, SB), i32),
            pltpu.VMEM((1, SB), i32),
            pltpu.VMEM((1, SB), i32),
            pltpu.VMEM((1, SB), i32),
            pltpu.VMEM((1, SB), i32),
            pltpu.VMEM((1, SB), i32),
            pltpu.VMEM((SB,), f32),
            pltpu.VMEM((SB,), f32),
            pltpu.VMEM((SB, E), f32),
            pltpu.VMEM((SB, E), f32),
            pltpu.VMEM((RPS, E), f32),
            pltpu.VMEM_SHARED((N, E), f32),
            pltpu.SemaphoreType.DMA,
            pltpu.SemaphoreType.DMA,
            pltpu.SemaphoreType.DMA,
            pltpu.SemaphoreType.DMA,
            pltpu.SemaphoreType.DMA,
            pltpu.SemaphoreType.DMA,
        ],
    )


def _make_k2b(N, E, NCOLS):
    NR = N * E // NCOLS

    def body(p_ref, b_ref, o_ref):
        o_ref[...] = jnp.maximum(p_ref[0] + p_ref[1] + b_ref[...], 0.0)

    return pl.pallas_call(
        body,
        out_shape=jax.ShapeDtypeStruct((NR, NCOLS), jnp.float32),
    )


def _make_k3(T, N, RP, E, SB, TR, BLK):
    NQ = 4
    QN = (RP // NQ) * N
    CH3 = T // NS
    NSUB = CH3 // SB
    ZR = (QN + TR) // NS
    ZB = ZR // 4
    CSB = SB + BLK + 2 * L

    def body(obj_h, degkey_h, hrelu_h,
             s2_h,
             ob0, ob1, dk0, dk1, blk0, blk1, hrows_v, zero_v, s2_s,
             semo0, semo1, semw0, semw1):
        cid = lax.axis_index("c")
        sid = lax.axis_index("s")
        iota = lax.iota(jnp.int32, L)
        ob = [ob0, ob1]
        dk = [dk0, dk1]
        blk = [blk0, blk1]
        semo = [semo0, semo1]
        semw = [semw0, semw1]

        for p in range(NQ // NC):
            q = NC * p + cid
            rbase = q * QN

            def _z(i, c):
                zero_v[i, :] = jnp.zeros((L,), jnp.float32)
                return c
            lax.fori_loop(0, ZB, _z, 0)
            for z in range(4):
                pltpu.sync_copy(zero_v, s2_s.at[pl.ds(sid * ZR + z * ZB, ZB)])
            plsc.subcore_barrier()

            def _issue(i, b):
                off = sid * CH3 + i * SB
                pltpu.sync_copy(obj_h.at[pl.ds(off, SB)], ob[b].at[0])
                pltpu.sync_copy(degkey_h.at[pl.ds(off, SB)], dk[b].at[0])
                return pltpu.async_copy(hrelu_h.at[ob[b].at[0]],
                                        hrows_v.at[b], semo[b])

            pend = {0: _issue(0, 0)}
            wpend = {}
            for sc in range(NSUB):
                b = sc % 2
                pend.pop(sc).wait()
                if sc + 1 < NSUB:
                    if sc - 1 in wpend:
                        wpend.pop(sc - 1).wait()
                    pend[sc + 1] = _issue(sc + 1, (sc + 1) % 2)

                def _f(g, c):
                    o = g * L
                    kk = dk[b][0, pl.ds(o, L)] - rbase
                    m = (kk >= 0) & (kk < QN)
                    tr = QN + ((o + iota) & (TR - 1))
                    blk[b][0, pl.ds(o, L)] = jnp.where(m, kk, tr)
                    return c
                lax.fori_loop(0, SB // L, _f, 0)
                wpend[sc] = pltpu.async_copy(
                    hrows_v.at[b], s2_s.at[blk[b].at[0]], semw[b], add=True)
            for cp in wpend.values():
                cp.wait()

            plsc.subcore_barrier()

            @pl.when(sid == 0)
            def _():
                pltpu.sync_copy(s2_s.at[pl.ds(0, QN)], s2_h.at[q])
            plsc.subcore_barrier()

    i32, f32 = jnp.int32, jnp.float32
    return pl.kernel(
        body,
        out_type=jax.ShapeDtypeStruct((NQ, QN, E), f32),
        mesh=_mesh(),
        compiler_params=pltpu.CompilerParams(needs_layout_passes=False, use_tc_tiling_on_sc=False),
        scratch_types=[
            pltpu.VMEM((1, SB), i32),
            pltpu.VMEM((1, SB), i32),
            pltpu.VMEM((1, SB), i32),
            pltpu.VMEM((1, SB), i32),
            pltpu.VMEM((1, SB), i32),
            pltpu.VMEM((1, SB), i32),
            pltpu.VMEM((2, SB, E), f32),
            pltpu.VMEM((ZB, E), f32),
            pltpu.VMEM_SHARED((QN + TR, E), f32),
            pltpu.SemaphoreType.DMA,
            pltpu.SemaphoreType.DMA,
            pltpu.SemaphoreType.DMA,
            pltpu.SemaphoreType.DMA,
        ],
    )


def _make_k4(N, RP, E, C, NB):
    NQ = 4
    RQ = RP // NQ
    GRID = N // NB

    def body(s2_ref, deg_ref, rd_ref, w2_ref, b2_ref, o_ref):
        d = deg_ref[:, :RP] + deg_ref[:, RP:]
        scale = jnp.where(d > 0.0, rd_ref[...] / d, 0.0)
        acc = jnp.zeros((NB, C), jnp.float32)
        for r in range(RP):
            h2r = s2_ref[r // RQ, r % RQ] * scale[:, r][:, None]
            acc += jnp.dot(h2r, w2_ref[r],
                           preferred_element_type=jnp.float32)
        o_ref[...] = acc + b2_ref[...]

    return pl.pallas_call(
        body,
        grid=(GRID,),
        in_specs=[
            pl.BlockSpec((NQ, RQ, NB, E), lambda i: (0, 0, i, 0)),
            pl.BlockSpec((NB, NC * RP), lambda i: (i, 0)),
            pl.BlockSpec((1, RP), lambda i: (0, 0)),
            pl.BlockSpec((RP, E, C), lambda i: (0, 0, 0)),
            pl.BlockSpec((1, C), lambda i: (0, 0)),
        ],
        out_specs=pl.BlockSpec((NB, C), lambda i: (i, 0)),
        out_shape=jax.ShapeDtypeStruct((N, C), jnp.float32),
    )


def kernel(weights1, weights2, bias1, bias2, relation_embeddings, row_indices,
           col_indices, hor_indices, ver_indices, nt):
    RP, N, E = weights1.shape
    C = weights2.shape[2]
    T = row_indices.shape[0]
    nt_s = hor_indices.shape[0] // RP

    CH = T // NW
    WIN = CH + L
    SB = 2000
    TR = 2048
    NB = 1000

    fr = hor_indices[:nt_s, 0]
    to_ = hor_indices[:nt_s, 1]
    frp = jnp.pad(fr, (0, WIN + 8))
    top = jnp.pad(to_, (0, WIN + 8))
    rdiag = jnp.diagonal(relation_embeddings).astype(jnp.float32)
    w1f = weights1.reshape(RP * N, E)

    k1 = _make_k1(T, N, RP, nt_s + WIN + 8, CH, WIN)
    subj, obj, degkey, w1key, deg = k1(
        row_indices, col_indices, frp, top, rdiag)

    k1c = _make_k1c(N, RP)
    scale = k1c(deg.reshape(NC, RP, N), rdiag.reshape(RP, 1)).reshape(RP * N)

    k2 = _make_k2(T, N, RP, E, CH, SB)
    hpart = k2(subj, w1key, degkey, scale, w1f)

    k2b = _make_k2b(N, E, 128)
    btile = jnp.tile(bias1, 128 // E).reshape(1, 128)
    hrelu = k2b(hpart.reshape(NC, N * E // 128, 128), btile).reshape(N, E)

    k3 = _make_k3(T, N, RP, E, SB, TR, 256)
    s2 = k3(obj, degkey, hrelu)

    k4 = _make_k4(N, RP, E, C, NB)
    degt = jnp.transpose(deg.reshape(NC * RP, N))
    out = k4(s2.reshape(4, RP // 4, N, E),
             degt,
             rdiag.reshape(1, RP),
             weights2,
             bias2.reshape(1, C))
    return out
